# baseline (speedup 1.0000x reference)
import functools

import numpy as np
import jax
import jax.numpy as jnp
from jax import lax
from jax.experimental import pallas as pl
from jax.experimental.pallas import tpu as pltpu
from jax.experimental.pallas import tpu_sc as plsc

_CB = 8
_B, _C, _H, _W = 4, 96, 512, 512
_NLOC = 64
_NWORK = 16
_CPW = (_B * _NLOC) // _NWORK
_CL = 128
_VL = 16

_chunks, _lanes = [], []
for _b in range(_B):
    for _c in range(_NLOC):
        _flat = (((_b * _C + _c) * _H + (7 * _c) % _H) * _W + (13 * _c) % _W)
        _chunks.append(_flat // _CL)
        _lanes.append(_flat % _CL)
_CHUNK_TAB = np.asarray(_chunks, np.int32)
_LANE_TAB = np.broadcast_to(
    np.asarray(_lanes, np.int32)[:, None], (_B * _NLOC, _VL)).copy()


def _copy_body(x_ref, o_ref):
    o_ref[...] = x_ref[...]


_sc_mesh = plsc.VectorSubcoreMesh(core_axis_name="c", subcore_axis_name="s")


@functools.partial(
    pl.kernel,
    out_type=(),
    mesh=_sc_mesh,
    scratch_types=[
        pltpu.VMEM((_CPW,), jnp.int32),
        pltpu.VMEM((_CPW, _VL), jnp.int32),
        pltpu.VMEM((_CPW, _CL), jnp.float32),
        pltpu.SemaphoreType.DMA,
    ],
)
def _sc_scatter(out_hbm, x_hbm, idx_hbm, lane_hbm, idx_v, lane_v, chunks_v,
                sem):
    nc = 2
    wid = lax.axis_index("s") * nc + lax.axis_index("c")

    @pl.when(wid < _NWORK)
    def _():
        base = wid * _CPW
        pltpu.sync_copy(idx_hbm.at[pl.ds(base, _CPW)], idx_v)
        pltpu.sync_copy(lane_hbm.at[pl.ds(base, _CPW), :], lane_v)
        pltpu.async_copy(x_hbm.at[idx_v], chunks_v, sem).wait()
        lane = lax.iota(jnp.int32, _VL)
        for j in range(_CPW):
            for k in range(_CL // _VL):
                pos = lane + k * _VL
                sl = pl.ds(k * _VL, _VL)
                chunks_v[j, sl] = jnp.where(
                    pos == lane_v[j, :], 0.0, chunks_v[j, sl])
        pltpu.async_copy(chunks_v, out_hbm.at[idx_v], sem).wait()


def kernel(X):
    B, C, H, W = X.shape
    n = B * C * H * W
    cp = pl.pallas_call(
        _copy_body,
        grid=(B * C // _CB,),
        in_specs=[pl.BlockSpec((_CB, H, W), lambda i: (i, 0, 0))],
        out_specs=pl.BlockSpec((_CB, H, W), lambda i: (i, 0, 0)),
        out_shape=jax.ShapeDtypeStruct((B * C, H, W), X.dtype),
    )(X.reshape(B * C, H, W))
    out_ref = jax.new_ref(cp.reshape(n // _CL, _CL))
    _sc_scatter(out_ref, X.reshape(n // _CL, _CL),
                jnp.asarray(_CHUNK_TAB), jnp.asarray(_LANE_TAB))
    return out_ref[...].reshape(B, C, H, W)


# speedup vs baseline: 1.1621x; 1.1621x over previous
import numpy as np
import jax
import jax.numpy as jnp
from jax import lax
from jax.experimental import pallas as pl
from jax.experimental.pallas import tpu as pltpu
from jax.experimental.pallas import tpu_sc as plsc
from jax._src.pallas import mpmd as _pl_mpmd

_CB = 8
_B, _C, _H, _W = 4, 96, 512, 512
_NLOC = 64
_NWORK = 16
_CPW = (_B * _NLOC) // _NWORK
_CL = 128
_VL = 16

_chunks, _lanes = [], []
for _b in range(_B):
    for _c in range(_NLOC):
        _flat = (((_b * _C + _c) * _H + (7 * _c) % _H) * _W + (13 * _c) % _W)
        _chunks.append(_flat // _CL)
        _lanes.append(_flat % _CL)
_CHUNK_TAB = np.asarray(_chunks, np.int32)
_LANE_TAB = np.broadcast_to(
    np.asarray(_lanes, np.int32)[:, None], (_B * _NLOC, _VL)).copy()


def _copy_body(x_ref, o_ref):
    o_ref[...] = x_ref[...]


_sc_mesh = plsc.VectorSubcoreMesh(core_axis_name="c", subcore_axis_name="s")


_N_CHUNKS = (_B * _C * _H * _W) // _CL


def _sc_scatter_body(cp_hbm, x_hbm, idx_hbm, lane_hbm, out_hbm,
                     idx_v, lane_v, chunks_v, sem):
    del cp_hbm
    nc = 2
    wid = lax.axis_index("s") * nc + lax.axis_index("c")

    @pl.when(wid < _NWORK)
    def _():
        base = wid * _CPW
        pltpu.sync_copy(idx_hbm.at[pl.ds(base, _CPW)], idx_v)
        pltpu.sync_copy(lane_hbm.at[pl.ds(base, _CPW), :], lane_v)
        pltpu.async_copy(x_hbm.at[idx_v], chunks_v, sem).wait()
        lane = lax.iota(jnp.int32, _VL)
        for j in range(_CPW):
            for k in range(_CL // _VL):
                pos = lane + k * _VL
                sl = pl.ds(k * _VL, _VL)
                chunks_v[j, sl] = jnp.where(
                    pos == lane_v[j, :], 0.0, chunks_v[j, sl])
        pltpu.async_copy(chunks_v, out_hbm.at[idx_v], sem).wait()


_sc_scatter = _pl_mpmd._mpmd_map(
    [(_sc_mesh, _sc_scatter_body)],
    jax.ShapeDtypeStruct((_N_CHUNKS, _CL), jnp.float32),
    input_output_aliases={0: 0},
    scratch_types=[
        pltpu.VMEM((_CPW,), jnp.int32),
        pltpu.VMEM((_CPW, _VL), jnp.int32),
        pltpu.VMEM((_CPW, _CL), jnp.float32),
        pltpu.SemaphoreType.DMA,
    ],
    compiler_params=pltpu.CompilerParams(use_tc_tiling_on_sc=False),
)


def kernel(X):
    B, C, H, W = X.shape
    n = B * C * H * W
    cp = pl.pallas_call(
        _copy_body,
        grid=(B * C // _CB,),
        in_specs=[pl.BlockSpec((_CB, H, W), lambda i: (i, 0, 0))],
        out_specs=pl.BlockSpec((_CB, H, W), lambda i: (i, 0, 0)),
        out_shape=jax.ShapeDtypeStruct((B * C, H, W), X.dtype),
    )(X.reshape(B * C, H, W))
    out = _sc_scatter(cp.reshape(n // _CL, _CL), X.reshape(n // _CL, _CL),
                      jnp.asarray(_CHUNK_TAB), jnp.asarray(_LANE_TAB))
    return out.reshape(B, C, H, W)
